# Initial kernel scaffold; baseline (speedup 1.0000x reference)
import jax
import jax.numpy as jnp
from jax.experimental import pallas as pl


def kernel(V, X, Wv, bv, Wf, bf):
    raise NotImplementedError("write your pallas kernel here")


# speedup vs baseline: 3.9357x; 3.9357x over previous
import functools

import jax
import jax.numpy as jnp
from jax import lax
from jax.experimental import pallas as pl
from jax.experimental.pallas import tpu as pltpu
from jax.experimental.pallas import tpu_sc as plsc

N1, D1 = 16384, 1024
N2, D2 = 4096, 1024
TK = 3
EPS = 1e-8

BM = 1024


def _proj_body(v_ref, wv_ref, bv_ref, p_ref, kn_ref):
    p = lax.dot_general(v_ref[...], wv_ref[...], (((1,), (1,)), ((), ())),
                        preferred_element_type=jnp.float32)
    p = p + bv_ref[...]
    nrm = jnp.sqrt(jnp.sum(p * p, axis=1, keepdims=True))
    p_ref[...] = p
    kn_ref[...] = p / (nrm + EPS)


def _project(V, Wv, bv2):
    return pl.pallas_call(
        _proj_body,
        grid=(N1 // BM,),
        in_specs=[
            pl.BlockSpec((BM, D1), lambda i: (i, 0)),
            pl.BlockSpec((D2, D1), lambda i: (0, 0)),
            pl.BlockSpec((1, D2), lambda i: (0, 0)),
        ],
        out_specs=[
            pl.BlockSpec((BM, D2), lambda i: (i, 0)),
            pl.BlockSpec((BM, D2), lambda i: (i, 0)),
        ],
        out_shape=[
            jax.ShapeDtypeStruct((N1, D2), jnp.float32),
            jax.ShapeDtypeStruct((N1, D2), jnp.float32),
        ],
    )(V, Wv, bv2)


BQ = 1024
CK = 1024

_NEG = float('-inf')
_BIGI = 1 << 30


def _topk_body(x_ref, kn_ref, o0_ref, o1_ref, o2_ref,
               qn_scr, v0s, v1s, v2s, i0s, i1s, i2s):
    j = pl.program_id(1)
    nj = pl.num_programs(1)

    @pl.when(j == 0)
    def _init():
        x = x_ref[...]
        nrm = jnp.sqrt(jnp.sum(x * x, axis=1, keepdims=True))
        qn_scr[...] = x / (nrm + EPS)
        neg = jnp.full((BQ, 1), _NEG, jnp.float32)
        v0s[...] = neg
        v1s[...] = neg
        v2s[...] = neg
        zero = jnp.zeros((BQ, 1), jnp.int32)
        i0s[...] = zero
        i1s[...] = zero
        i2s[...] = zero

    s = lax.dot_general(qn_scr[...], kn_ref[...], (((1,), (1,)), ((), ())),
                        preferred_element_type=jnp.float32)
    off = j * CK
    colid = lax.broadcasted_iota(jnp.int32, (BQ, CK), 1)
    v0, v1, v2 = v0s[...], v1s[...], v2s[...]
    i0, i1, i2 = i0s[...], i1s[...], i2s[...]
    for _ in range(TK):
        m = jnp.max(s, axis=1, keepdims=True)
        a_loc = jnp.min(jnp.where(s == m, colid, _BIGI), axis=1, keepdims=True)
        s = jnp.where(colid == a_loc, _NEG, s)
        a = a_loc + off
        b0 = m > v0
        b1 = m > v1
        b2 = m > v2
        i2n = jnp.where(b1, i1, jnp.where(b2, a, i2))
        v2n = jnp.where(b1, v1, jnp.where(b2, m, v2))
        i1n = jnp.where(b0, i0, jnp.where(b1, a, i1))
        v1n = jnp.where(b0, v0, jnp.where(b1, m, v1))
        i0n = jnp.where(b0, a, i0)
        v0n = jnp.where(b0, m, v0)
        v0, v1, v2, i0, i1, i2 = v0n, v1n, v2n, i0n, i1n, i2n
    v0s[...], v1s[...], v2s[...] = v0, v1, v2
    i0s[...], i1s[...], i2s[...] = i0, i1, i2

    @pl.when(j == nj - 1)
    def _emit():
        o0_ref[...] = i0s[...]
        o1_ref[...] = i1s[...]
        o2_ref[...] = i2s[...]


def _top3(X, KN):
    return pl.pallas_call(
        _topk_body,
        grid=(N2 // BQ, N1 // CK),
        in_specs=[
            pl.BlockSpec((BQ, D1), lambda i, j: (i, 0)),
            pl.BlockSpec((CK, D1), lambda i, j: (j, 0)),
        ],
        out_specs=[
            pl.BlockSpec((BQ, 1), lambda i, j: (i, 0)),
            pl.BlockSpec((BQ, 1), lambda i, j: (i, 0)),
            pl.BlockSpec((BQ, 1), lambda i, j: (i, 0)),
        ],
        out_shape=[
            jax.ShapeDtypeStruct((N2, 1), jnp.int32),
            jax.ShapeDtypeStruct((N2, 1), jnp.int32),
            jax.ShapeDtypeStruct((N2, 1), jnp.int32),
        ],
        scratch_shapes=[
            pltpu.VMEM((BQ, D1), jnp.float32),
            pltpu.VMEM((BQ, 1), jnp.float32),
            pltpu.VMEM((BQ, 1), jnp.float32),
            pltpu.VMEM((BQ, 1), jnp.float32),
            pltpu.VMEM((BQ, 1), jnp.int32),
            pltpu.VMEM((BQ, 1), jnp.int32),
            pltpu.VMEM((BQ, 1), jnp.int32),
        ],
    )(X, KN)


NW = 32
QPW = N2 // NW
CQ = 16
NCH = QPW // CQ
ROWS = CQ * TK


def _fuse_body(p_hbm, idx_hbm, w_hbm, out_hbm,
               w_v, idx_a, idx_b, rows_a, rows_b, out_v, sem_a, sem_b):
    cid = lax.axis_index("c")
    sid = lax.axis_index("s")
    wid = sid * 2 + cid
    qbase = wid * QPW
    pltpu.sync_copy(w_hbm, w_v)
    w0 = w_v[0]
    w1 = w_v[1]
    w2 = w_v[2]
    wb = w_v[3]
    idx_bufs = (idx_a, idx_b)
    row_bufs = (rows_a, rows_b)
    sems = (sem_a, sem_b)
    handles = [None] * NCH

    def fire(c):
        sl = c & 1
        pltpu.sync_copy(idx_hbm.at[pl.ds((qbase + c * CQ) * TK, ROWS)],
                        idx_bufs[sl])
        handles[c] = pltpu.async_copy(p_hbm.at[idx_bufs[sl]], row_bufs[sl],
                                      sems[sl])

    fire(0)
    for c in range(NCH):
        if c + 1 < NCH:
            fire(c + 1)
        handles[c].wait()
        rows = row_bufs[c & 1]

        def dbody(d, _):
            sl = pl.ds(d * 16, 16)
            for q in range(CQ):
                r0 = rows[TK * q, sl]
                r1 = rows[TK * q + 1, sl]
                r2 = rows[TK * q + 2, sl]
                out_v[q, sl] = (w0 * r0 + w1 * r1 + w2 * r2) + wb
            return 0

        lax.fori_loop(0, D2 // 16, dbody, 0)
        pltpu.sync_copy(out_v, out_hbm.at[pl.ds(qbase + c * CQ, CQ)])


def _sc_fuse(P, idx_flat, wmat):
    mesh = plsc.VectorSubcoreMesh(core_axis_name="c", subcore_axis_name="s")
    f = functools.partial(
        pl.kernel,
        out_type=jax.ShapeDtypeStruct((N2, D2), jnp.float32),
        mesh=mesh,
        scratch_types=[
            pltpu.VMEM((4, 16), jnp.float32),
            pltpu.VMEM((ROWS,), jnp.int32),
            pltpu.VMEM((ROWS,), jnp.int32),
            pltpu.VMEM((ROWS, D2), jnp.float32),
            pltpu.VMEM((ROWS, D2), jnp.float32),
            pltpu.VMEM((CQ, D2), jnp.float32),
            pltpu.SemaphoreType.DMA,
            pltpu.SemaphoreType.DMA,
        ],
    )(_fuse_body)
    return f(P, idx_flat, wmat)


def kernel(V, X, Wv, bv, Wf, bf):
    bv2 = bv.reshape(1, D2)
    P, KN = _project(V, Wv, bv2)
    o0, o1, o2 = _top3(X, KN)
    idx_flat = jnp.concatenate([o0, o1, o2], axis=1).reshape(-1)
    wmat = jnp.concatenate([Wf[0], bf]).reshape(4, 1) * jnp.ones((1, 16), jnp.float32)
    fused = _sc_fuse(P, idx_flat, wmat)
    return jnp.concatenate([fused, X], axis=0)


# speedup vs baseline: 4.5316x; 1.1514x over previous
import functools

import jax
import jax.numpy as jnp
from jax import lax
from jax.experimental import pallas as pl
from jax.experimental.pallas import tpu as pltpu
from jax.experimental.pallas import tpu_sc as plsc

N1, D1 = 16384, 1024
N2, D2 = 4096, 1024
TK = 3
EPS = 1e-8

BM = 1024


def _proj_body(v_ref, wv_ref, bv_ref, p_ref, kn_ref):
    p = lax.dot_general(v_ref[...], wv_ref[...], (((1,), (1,)), ((), ())),
                        preferred_element_type=jnp.float32)
    p = p + bv_ref[...]
    nrm = jnp.sqrt(jnp.sum(p * p, axis=1, keepdims=True))
    p_ref[...] = p
    kn_ref[...] = p / (nrm + EPS)


def _project(V, Wv, bv2):
    return pl.pallas_call(
        _proj_body,
        grid=(N1 // BM,),
        in_specs=[
            pl.BlockSpec((BM, D1), lambda i: (i, 0)),
            pl.BlockSpec((D2, D1), lambda i: (0, 0)),
            pl.BlockSpec((1, D2), lambda i: (0, 0)),
        ],
        out_specs=[
            pl.BlockSpec((BM, D2), lambda i: (i, 0)),
            pl.BlockSpec((BM, D2), lambda i: (i, 0)),
        ],
        out_shape=[
            jax.ShapeDtypeStruct((N1, D2), jnp.float32),
            jax.ShapeDtypeStruct((N1, D2), jnp.float32),
        ],
    )(V, Wv, bv2)


BQ = 1024
CK = 1024

_NEG = float('-inf')
_BIGF = float(1 << 26)


_NT = N1 // CK // 2


def _extract3(s_ref, off, v0s, v1s, v2s, i0s, i1s, i2s):
    s = s_ref[...]
    colid = lax.broadcasted_iota(jnp.int32, (BQ, CK), 1).astype(jnp.float32)
    v0, v1, v2 = v0s[...], v1s[...], v2s[...]
    i0, i1, i2 = i0s[...], i1s[...], i2s[...]
    for _ in range(TK):
        m = jnp.max(s, axis=1, keepdims=True)
        a_loc = jnp.min(jnp.where(s == m, colid, _BIGF), axis=1, keepdims=True)
        s = jnp.where(colid == a_loc, _NEG, s)
        a = a_loc + off
        b0 = m > v0
        b1 = m > v1
        b2 = m > v2
        i2n = jnp.where(b1, i1, jnp.where(b2, a, i2))
        v2n = jnp.where(b1, v1, jnp.where(b2, m, v2))
        i1n = jnp.where(b0, i0, jnp.where(b1, a, i1))
        v1n = jnp.where(b0, v0, jnp.where(b1, m, v1))
        i0n = jnp.where(b0, a, i0)
        v0n = jnp.where(b0, m, v0)
        v0, v1, v2, i0, i1, i2 = v0n, v1n, v2n, i0n, i1n, i2n
    v0s[...], v1s[...], v2s[...] = v0, v1, v2
    i0s[...], i1s[...], i2s[...] = i0, i1, i2


def _topk_body(x_ref, kna_ref, knb_ref, o0_ref, o1_ref, o2_ref,
               qn_scr, s_a, s_b, v0s, v1s, v2s, i0s, i1s, i2s):
    t = pl.program_id(1)

    @pl.when(t == 0)
    def _init():
        x = x_ref[...]
        nrm = jnp.sqrt(jnp.sum(x * x, axis=1, keepdims=True))
        qn_scr[...] = x / (nrm + EPS)
        neg = jnp.full((BQ, 1), _NEG, jnp.float32)
        v0s[...] = neg
        v1s[...] = neg
        v2s[...] = neg
        zero = jnp.zeros((BQ, 1), jnp.float32)
        i0s[...] = zero
        i1s[...] = zero
        i2s[...] = zero
        s_b[...] = jnp.full((BQ, CK), _NEG, jnp.float32)

    off = 2 * t * CK
    s_a[...] = lax.dot_general(
        qn_scr[...], kna_ref[...], (((1,), (1,)), ((), ())),
        preferred_element_type=jnp.float32)
    _extract3(s_b, off - CK, v0s, v1s, v2s, i0s, i1s, i2s)
    s_b[...] = lax.dot_general(
        qn_scr[...], knb_ref[...], (((1,), (1,)), ((), ())),
        preferred_element_type=jnp.float32)
    _extract3(s_a, off, v0s, v1s, v2s, i0s, i1s, i2s)

    @pl.when(t == _NT - 1)
    def _tail():
        _extract3(s_b, off + CK, v0s, v1s, v2s, i0s, i1s, i2s)
        o0_ref[...] = i0s[...].astype(jnp.int32)
        o1_ref[...] = i1s[...].astype(jnp.int32)
        o2_ref[...] = i2s[...].astype(jnp.int32)


def _top3(X, KN):
    return pl.pallas_call(
        _topk_body,
        grid=(N2 // BQ, _NT),
        in_specs=[
            pl.BlockSpec((BQ, D1), lambda i, t: (i, 0)),
            pl.BlockSpec((CK, D1), lambda i, t: (2 * t, 0)),
            pl.BlockSpec((CK, D1), lambda i, t: (2 * t + 1, 0)),
        ],
        out_specs=[
            pl.BlockSpec((BQ, 1), lambda i, t: (i, 0)),
            pl.BlockSpec((BQ, 1), lambda i, t: (i, 0)),
            pl.BlockSpec((BQ, 1), lambda i, t: (i, 0)),
        ],
        out_shape=[
            jax.ShapeDtypeStruct((N2, 1), jnp.int32),
            jax.ShapeDtypeStruct((N2, 1), jnp.int32),
            jax.ShapeDtypeStruct((N2, 1), jnp.int32),
        ],
        scratch_shapes=[
            pltpu.VMEM((BQ, D1), jnp.float32),
            pltpu.VMEM((BQ, CK), jnp.float32),
            pltpu.VMEM((BQ, CK), jnp.float32),
            pltpu.VMEM((BQ, 1), jnp.float32),
            pltpu.VMEM((BQ, 1), jnp.float32),
            pltpu.VMEM((BQ, 1), jnp.float32),
            pltpu.VMEM((BQ, 1), jnp.float32),
            pltpu.VMEM((BQ, 1), jnp.float32),
            pltpu.VMEM((BQ, 1), jnp.float32),
        ],
    )(X, KN, KN)


NW = 32
QPW = N2 // NW
CQ = 16
NCH = QPW // CQ
ROWS = CQ * TK


def _fuse_body(p_hbm, idx_hbm, w_hbm, out_hbm,
               w_v, idx_a, idx_b, rows_a, rows_b, out_v, sem_a, sem_b):
    cid = lax.axis_index("c")
    sid = lax.axis_index("s")
    wid = sid * 2 + cid
    qbase = wid * QPW
    pltpu.sync_copy(w_hbm, w_v)
    w0 = w_v[0]
    w1 = w_v[1]
    w2 = w_v[2]
    wb = w_v[3]
    idx_bufs = (idx_a, idx_b)
    row_bufs = (rows_a, rows_b)
    sems = (sem_a, sem_b)
    handles = [None] * NCH

    def fire(c):
        sl = c & 1
        pltpu.sync_copy(idx_hbm.at[pl.ds((qbase + c * CQ) * TK, ROWS)],
                        idx_bufs[sl])
        handles[c] = pltpu.async_copy(p_hbm.at[idx_bufs[sl]], row_bufs[sl],
                                      sems[sl])

    fire(0)
    for c in range(NCH):
        if c + 1 < NCH:
            fire(c + 1)
        handles[c].wait()
        rows = row_bufs[c & 1]

        def dbody(d, _):
            sl = pl.ds(d * 16, 16)
            for q in range(CQ):
                r0 = rows[TK * q, sl]
                r1 = rows[TK * q + 1, sl]
                r2 = rows[TK * q + 2, sl]
                out_v[q, sl] = (w0 * r0 + w1 * r1 + w2 * r2) + wb
            return 0

        lax.fori_loop(0, D2 // 16, dbody, 0)
        pltpu.sync_copy(out_v, out_hbm.at[pl.ds(qbase + c * CQ, CQ)])


def _sc_fuse(P, idx_flat, wmat):
    mesh = plsc.VectorSubcoreMesh(core_axis_name="c", subcore_axis_name="s")
    f = functools.partial(
        pl.kernel,
        out_type=jax.ShapeDtypeStruct((N2, D2), jnp.float32),
        mesh=mesh,
        scratch_types=[
            pltpu.VMEM((4, 16), jnp.float32),
            pltpu.VMEM((ROWS,), jnp.int32),
            pltpu.VMEM((ROWS,), jnp.int32),
            pltpu.VMEM((ROWS, D2), jnp.float32),
            pltpu.VMEM((ROWS, D2), jnp.float32),
            pltpu.VMEM((CQ, D2), jnp.float32),
            pltpu.SemaphoreType.DMA,
            pltpu.SemaphoreType.DMA,
        ],
    )(_fuse_body)
    return f(P, idx_flat, wmat)


def kernel(V, X, Wv, bv, Wf, bf):
    bv2 = bv.reshape(1, D2)
    P, KN = _project(V, Wv, bv2)
    o0, o1, o2 = _top3(X, KN)
    idx_flat = jnp.concatenate([o0, o1, o2], axis=1).reshape(-1)
    wmat = jnp.concatenate([Wf[0], bf]).reshape(4, 1) * jnp.ones((1, 16), jnp.float32)
    fused = _sc_fuse(P, idx_flat, wmat)
    return jnp.concatenate([fused, X], axis=0)
